# Initial kernel scaffold; baseline (speedup 1.0000x reference)
import jax
import jax.numpy as jnp
from jax.experimental import pallas as pl


def kernel(h, edge_index, W0, b0, W1, b1, W2, b2, g0, be0, g1, be1, g2, be2, fcW, fcb):
    raise NotImplementedError("write your pallas kernel here")


# speedup vs baseline: 7.2347x; 7.2347x over previous
import functools

import jax
import jax.numpy as jnp
from jax import lax
from jax.experimental import pallas as pl
from jax.experimental.pallas import tpu as pltpu
from jax.experimental.pallas import tpu_sc as plsc

_WTAB = [0.7, 0.9, 0.7, 0.9, 0.3, 0.7, 0.3, 0.9, 0.3, 0.3, 0.9, 0.7, 0.1,
         0.9, 0.5, 0.9, 0.5, 0.5, 0.1, 0.3, 0.7, 0.9, 0.9, 0.9, 0.9, 0.9]
_EPS = 1e-5


def _fill_rows(buf, nrows, ncols, value):
    vec = jnp.full((16,), value, jnp.float32)

    def row(i, carry):
        for c in range(ncols // 16):
            buf[i, pl.ds(c * 16, 16)] = vec
        return carry

    lax.fori_loop(0, nrows, row, 0)


@functools.lru_cache(maxsize=None)
def _make_sc_agg(n_acc, erows, dh, chr_):
    rt = erows // 16
    nch = n_acc // 128
    zch = (nch + 15) // 16
    mesh = plsc.VectorSubcoreMesh(core_axis_name="c", subcore_axis_name="s")

    def body(xs0, xs1, srcp, dstp, out0, out1, sidx, didx, rows, zbuf, acc, sem):
        cid = lax.axis_index("c")
        sid = lax.axis_index("s")

        _fill_rows(zbuf, 128, dh, 0.0)

        def zacc(j, carry):
            ch = sid + 16 * j

            @pl.when(ch < nch)
            def _():
                pltpu.sync_copy(zbuf, acc.at[pl.ds(ch * 128, 128)])

            return carry

        lax.fori_loop(0, zch, zacc, 0)
        plsc.subcore_barrier()

        def run(xs, out):
            def chunk(g, carry):
                base = sid * rt + g * chr_
                pltpu.sync_copy(srcp.at[pl.ds(base, chr_)], sidx)
                pltpu.sync_copy(dstp.at[pl.ds(base, chr_)], didx)
                cps = [pltpu.async_copy(xs.at[sidx.at[j]], rows.at[j], sem)
                       for j in range(chr_)]
                for cp in cps:
                    cp.wait()
                for j in range(chr_):
                    pltpu.sync_copy(rows.at[j], acc.at[didx.at[j]], add=True)
                return carry

            lax.fori_loop(0, rt // chr_, chunk, 0)
            plsc.subcore_barrier()

            def wb(j, carry):
                ch = sid + 16 * j

                @pl.when(ch < nch)
                def _():
                    pltpu.sync_copy(acc.at[pl.ds(ch * 128, 128)],
                                    out.at[pl.ds(ch * 128, 128)])

                return carry

            lax.fori_loop(0, zch, wb, 0)

        @pl.when(cid == 0)
        def _():
            run(xs0, out0)

        @pl.when(cid == 1)
        def _():
            run(xs1, out1)

    return pl.kernel(
        body,
        mesh=mesh,
        out_type=[jax.ShapeDtypeStruct((n_acc, dh), jnp.float32),
                  jax.ShapeDtypeStruct((n_acc, dh), jnp.float32)],
        scratch_types=[
            pltpu.VMEM((chr_, 128), jnp.int32),
            pltpu.VMEM((chr_, 128), jnp.int32),
            pltpu.VMEM((chr_, 128, dh), jnp.float32),
            pltpu.VMEM((128, dh), jnp.float32),
            pltpu.VMEM_SHARED((n_acc, dh), jnp.float32),
            pltpu.SemaphoreType.DMA,
        ],
        compiler_params=pltpu.CompilerParams(use_tc_tiling_on_sc=False),
    )


@functools.lru_cache(maxsize=None)
def _make_sc_deg(n_acc, erows):
    rt = erows // 16
    nch = n_acc // 256
    zch = (nch + 15) // 16
    mesh = plsc.VectorSubcoreMesh(core_axis_name="c", subcore_axis_name="s")

    def body(srcd, dstp, out_o, out_i, idxb, onesb, zbuf, acc):
        cid = lax.axis_index("c")
        sid = lax.axis_index("s")

        _fill_rows(zbuf, 256, 16, 0.0)
        _fill_rows(onesb, 128, 16, 1.0)

        def zacc(j, carry):
            ch = sid + 16 * j

            @pl.when(ch < nch)
            def _():
                pltpu.sync_copy(zbuf, acc.at[pl.ds(ch * 256, 256)])

            return carry

        lax.fori_loop(0, zch, zacc, 0)
        plsc.subcore_barrier()

        def run(idxs, out):
            def chunk(g, carry):
                base = sid * rt + g * 8
                pltpu.sync_copy(idxs.at[pl.ds(base, 8)], idxb)
                for j in range(8):
                    pltpu.sync_copy(onesb, acc.at[idxb.at[j]], add=True)
                return carry

            lax.fori_loop(0, rt // 8, chunk, 0)
            plsc.subcore_barrier()

            def wb(j, carry):
                ch = sid + 16 * j

                @pl.when(ch < nch)
                def _():
                    pltpu.sync_copy(acc.at[pl.ds(ch * 256, 256)],
                                    out.at[pl.ds(ch * 256, 256)])

                return carry

            lax.fori_loop(0, zch, wb, 0)

        @pl.when(cid == 0)
        def _():
            run(srcd, out_o)

        @pl.when(cid == 1)
        def _():
            run(dstp, out_i)

    return pl.kernel(
        body,
        mesh=mesh,
        out_type=[jax.ShapeDtypeStruct((n_acc, 16), jnp.float32),
                  jax.ShapeDtypeStruct((n_acc, 16), jnp.float32)],
        scratch_types=[
            pltpu.VMEM((8, 128), jnp.int32),
            pltpu.VMEM((128, 16), jnp.float32),
            pltpu.VMEM((256, 16), jnp.float32),
            pltpu.VMEM_SHARED((n_acc, 16), jnp.float32),
        ],
        compiler_params=pltpu.CompilerParams(use_tc_tiling_on_sc=False),
    )


def kernel(h, edge_index, W0, b0, W1, b1, W2, b2, g0, be0, g1, be1, g2, be2,
           fcW, fcb):
    n = h.shape[0]
    e = edge_index.shape[1]
    n_acc = ((n + 1 + 255) // 256) * 256
    epad = ((e + 16383) // 16384) * 16384
    erows = epad // 128

    src = edge_index[0]
    dst = edge_index[1]
    pad0 = jnp.zeros((epad - e,), jnp.int32)
    padn = jnp.full((epad - e,), n, jnp.int32)
    srcg = jnp.concatenate([src, pad0]).reshape(erows, 128)
    srcd = jnp.concatenate([src, padn]).reshape(erows, 128)
    dstp = jnp.concatenate([dst, padn]).reshape(erows, 128)

    deg_o, deg_i = _make_sc_deg(n_acc, erows)(srcd, dstp)
    deg_out = deg_o[:n, 0]
    deg_in = deg_i[:n, 0]
    norm_src = jnp.where(deg_out > 0, lax.rsqrt(jnp.maximum(deg_out, 1.0)), 0.0)
    norm_dst = jnp.where(deg_in > 0, lax.rsqrt(jnp.maximum(deg_in, 1.0)), 0.0)

    tab = jnp.array(_WTAB, dtype=jnp.float32)
    node_w = jnp.take(tab, jnp.argmax(h, axis=1))[:, None]

    x = h
    for li, (W, b, g, be) in enumerate([(W0, b0, g0, be0), (W1, b1, g1, be1),
                                        (W2, b2, g2, be2)]):
        xs = x * norm_src[:, None]
        d = xs.shape[1]
        dh = 16 if d <= 32 else 32
        if d < 2 * dh:
            xs = jnp.pad(xs, ((0, 0), (0, 2 * dh - d)))
        a0, a1 = _make_sc_agg(n_acc, erows, dh, 8 if dh == 16 else 4)(
            xs[:, :dh], xs[:, dh:], srcg, dstp)
        agg = jnp.concatenate([a0[:n], a1[:n]], axis=1)[:, :d]
        agg = agg * norm_dst[:, None]
        y = agg @ W + b
        mu = jnp.mean(y, axis=0)
        var = jnp.var(y, axis=0)
        y = (y - mu) / jnp.sqrt(var + _EPS) * g + be
        x = jax.nn.relu(y)
    out = x @ fcW + fcb
    return (out, node_w)
